# baseline (speedup 1.0000x reference)
import functools

import jax
import jax.numpy as jnp
from jax import lax
from jax.experimental import pallas as pl
from jax.experimental.pallas import tpu as pltpu
from jax.experimental.pallas import tpu_sc as plsc

_NC = 2
_NS = 16
_NW = _NC * _NS
_C = 64
_K = 20
_L = 16


@functools.cache
def _build(n_total, v, d):
    n_per_w = n_total // _NW
    n_chunks = n_per_w // _C
    n_blocks = n_chunks // _K
    mesh = plsc.VectorSubcoreMesh(core_axis_name="c", subcore_axis_name="s")

    @functools.partial(
        pl.kernel,
        out_type=jax.ShapeDtypeStruct((n_total, d), jnp.float32),
        mesh=mesh,
        scratch_types=[
            pltpu.VMEM((_K, _C), jnp.int32),
            pltpu.VMEM((_C, d), jnp.float32),
            pltpu.VMEM((_C, d), jnp.float32),
            pltpu.VMEM_SHARED((v, d), jnp.float32),
            pltpu.SemaphoreType.DMA,
            pltpu.SemaphoreType.DMA,
            pltpu.SemaphoreType.DMA,
            pltpu.SemaphoreType.DMA,
        ],
    )
    def gather_k(table_hbm, idx_hbm, out_hbm, iblk, row0, row1, table_sh,
                 sg0, sg1, ss0, ss1):
        s = lax.axis_index("s")
        wid = s * _NC + lax.axis_index("c")
        base = wid * n_per_w
        v_per_s = v // _NS
        pltpu.sync_copy(table_hbm.at[pl.ds(s * v_per_s, v_per_s)],
                        table_sh.at[pl.ds(s * v_per_s, v_per_s)])
        plsc.subcore_barrier()

        def fetch_rows(k, row, sg):
            for u in range(_C // _L):
                vec = iblk[k, pl.ds(u * _L, _L)]
                for l in range(_L):
                    pltpu.async_copy(
                        table_sh.at[pl.ds(vec[l], 1)],
                        row.at[pl.ds(u * _L + l, 1)], sg)

        def drain_rows(row, sg):
            for _ in range(_C):
                pltpu.make_async_copy(table_sh.at[pl.ds(0, 1)],
                                      row.at[pl.ds(0, 1)], sg).wait()

        def put(j, row, ss):
            pltpu.async_copy(row, out_hbm.at[pl.ds(base + j * _C, _C)], ss)

        def wait_s(row, ss):
            pltpu.make_async_copy(row, out_hbm.at[pl.ds(base, _C)],
                                  ss).wait()

        def block(b, carry):
            pltpu.sync_copy(idx_hbm.at[wid * n_blocks + b], iblk)
            cbase = b * _K

            def pair(g, c2):
                for k_off, row, sg, ss in ((0, row0, sg0, ss0),
                                           (1, row1, sg1, ss1)):
                    k = 2 * g + k_off

                    @pl.when(jnp.logical_or(b > 0, g > 0))
                    def _():
                        wait_s(row, ss)

                    fetch_rows(k, row, sg)
                    drain_rows(row, sg)
                    put(cbase + k, row, ss)
                return c2

            lax.fori_loop(0, _K // 2, pair, 0)
            return carry

        lax.fori_loop(0, n_blocks, block, 0)
        wait_s(row0, ss0)
        wait_s(row1, ss1)

    return gather_k


def kernel(x, table):
    b, h = x.shape
    v, d = table.shape
    n_total = b * h
    n_blocks = n_total // _NW // _C // _K
    idx = x.reshape(_NW * n_blocks, _K, _C)
    out = _build(n_total, v, d)(table, idx)
    return out.reshape(b, h, d)


# speedup vs baseline: 1.0264x; 1.0264x over previous
import functools

import jax
import jax.numpy as jnp
from jax import lax
from jax.experimental import pallas as pl
from jax.experimental.pallas import tpu as pltpu
from jax.experimental.pallas import tpu_sc as plsc

_NC = 2
_NS = 16
_NW = _NC * _NS
_C = 64
_K = 20
_L = 16


@functools.cache
def _build(n_total, v, d):
    n_per_w = n_total // _NW
    n_chunks = n_per_w // _C
    n_blocks = n_chunks // _K
    mesh = plsc.VectorSubcoreMesh(core_axis_name="c", subcore_axis_name="s")

    @functools.partial(
        pl.kernel,
        out_type=jax.ShapeDtypeStruct((n_total, d), jnp.float32),
        mesh=mesh,
        scratch_types=[
            pltpu.VMEM((_K, _C), jnp.int32),
            pltpu.VMEM((_C, d), jnp.float32),
            pltpu.VMEM((_C, d), jnp.float32),
            pltpu.VMEM_SHARED((v, d), jnp.float32),
            pltpu.SemaphoreType.DMA,
            pltpu.SemaphoreType.DMA,
            pltpu.SemaphoreType.DMA,
            pltpu.SemaphoreType.DMA,
        ],
    )
    def gather_k(table_hbm, idx_hbm, out_hbm, iblk, row0, row1, table_sh,
                 sg0, sg1, ss0, ss1):
        s = lax.axis_index("s")
        wid = s * _NC + lax.axis_index("c")
        base = wid * n_per_w
        v_per_s = v // _NS
        pltpu.sync_copy(table_hbm.at[pl.ds(s * v_per_s, v_per_s)],
                        table_sh.at[pl.ds(s * v_per_s, v_per_s)])
        plsc.subcore_barrier()

        def fetch_rows(k, row, sg):
            for u in range(_C // _L):
                vec = iblk[k, pl.ds(u * _L, _L)]
                for l in range(_L):
                    pltpu.async_copy(
                        table_sh.at[pl.ds(vec[l], 1)],
                        row.at[pl.ds(u * _L + l, 1)], sg)

        def drain_rows(row, sg):
            pltpu.make_async_copy(table_sh.at[pl.ds(0, _C)], row, sg).wait()

        def put(j, row, ss):
            pltpu.async_copy(row, out_hbm.at[pl.ds(base + j * _C, _C)], ss)

        def wait_s(row, ss):
            pltpu.make_async_copy(row, out_hbm.at[pl.ds(base, _C)],
                                  ss).wait()

        def block(b, carry):
            pltpu.sync_copy(idx_hbm.at[wid * n_blocks + b], iblk)
            cbase = b * _K

            def pair(g, c2):
                not_first = jnp.logical_or(b > 0, g > 0)

                @pl.when(not_first)
                def _():
                    wait_s(row0, ss0)

                fetch_rows(2 * g, row0, sg0)

                @pl.when(not_first)
                def _():
                    wait_s(row1, ss1)

                fetch_rows(2 * g + 1, row1, sg1)
                drain_rows(row0, sg0)
                put(cbase + 2 * g, row0, ss0)
                drain_rows(row1, sg1)
                put(cbase + 2 * g + 1, row1, ss1)
                return c2

            lax.fori_loop(0, _K // 2, pair, 0)
            return carry

        lax.fori_loop(0, n_blocks, block, 0)
        wait_s(row0, ss0)
        wait_s(row1, ss1)

    return gather_k


def kernel(x, table):
    b, h = x.shape
    v, d = table.shape
    n_total = b * h
    n_blocks = n_total // _NW // _C // _K
    idx = x.reshape(_NW * n_blocks, _K, _C)
    out = _build(n_total, v, d)(table, idx)
    return out.reshape(b, h, d)
